# Initial kernel scaffold; baseline (speedup 1.0000x reference)
import jax
import jax.numpy as jnp
from jax.experimental import pallas as pl


def kernel(x, edge_index, W1, att_src, att_dst, b1, W2, b2):
    raise NotImplementedError("write your pallas kernel here")


# speedup vs baseline: 16.6655x; 16.6655x over previous
import functools

import jax
import jax.numpy as jnp
from jax import lax
from jax.experimental import pallas as pl
from jax.experimental.pallas import tpu as pltpu
from jax.experimental.pallas import tpu_sc as plsc

N = 10000
E = 320000
IN_C = 128
HID = 128
OUT_C = 64

NC = 2
NS = 16
NW = NC * NS
K = 128
CHUNKS = 80
EPW = K * CHUNKS
E_PAD = EPW * NW
NPAD = 10112
STRIPE = NPAD // NS
RB = 25
R = N // RB


def _p1_body(x_ref, w1_ref, att2_ref, h_ref, a_ref):
    h = jax.lax.dot_general(x_ref[...], w1_ref[...], (((1,), (0,)), ((), ())),
                            preferred_element_type=jnp.float32)
    h_ref[...] = h
    a_ref[...] = jax.lax.dot_general(h, att2_ref[...], (((1,), (0,)), ((), ())),
                                     preferred_element_type=jnp.float32)


def _phase1(x, W1, att2):
    return pl.pallas_call(
        _p1_body,
        grid=(RB,),
        in_specs=[
            pl.BlockSpec((R, IN_C), lambda i: (i, 0)),
            pl.BlockSpec((IN_C, HID), lambda i: (0, 0)),
            pl.BlockSpec((HID, 2), lambda i: (0, 0)),
        ],
        out_specs=[
            pl.BlockSpec((R, HID), lambda i: (i, 0)),
            pl.BlockSpec((R, 2), lambda i: (i, 0)),
        ],
        out_shape=[
            jax.ShapeDtypeStruct((N, HID), jnp.float32),
            jax.ShapeDtypeStruct((N, 2), jnp.float32),
        ],
    )(x, W1, att2)


def _stripe_offsets():
    return list(range(0, STRIPE, K))


def _sc_body(h_hbm, a_hbm, src_hbm, dst_hbm,
             outp_hbm, denp_hbm,
             tab_v, sidx_v, didx_v, w_v, rows_v, denbuf_v,
             out_sh, den_sh,
             sem):
    c = lax.axis_index("c")
    s = lax.axis_index("s")
    wid = c * NS + s

    pltpu.sync_copy(a_hbm, tab_v.at[pl.ds(0, 2 * N)])
    z16 = jnp.zeros((16,), jnp.float32)
    tab_v[pl.ds(2 * N, 16)] = z16
    tab_v[pl.ds(2 * N + 16, 16)] = z16

    def _zrow(j, _):
        for g in range(HID // 16):
            rows_v[j, pl.ds(g * 16, 16)] = z16
        return 0
    lax.fori_loop(0, K, _zrow, 0)
    base = s * STRIPE
    for off in _stripe_offsets():
        sz = min(K, STRIPE - off)
        pltpu.sync_copy(rows_v.at[pl.ds(0, sz)], out_sh.at[pl.ds(base + off, sz)])
        pltpu.sync_copy(rows_v.at[0, pl.ds(0, sz)] if sz < K else rows_v.at[0],
                        den_sh.at[pl.ds(base + off, sz)])
    plsc.subcore_barrier()

    def _chunk(ci, _):
        ebase = wid * EPW + ci * K
        pltpu.sync_copy(src_hbm.at[pl.ds(ebase, K)], sidx_v)
        pltpu.sync_copy(dst_hbm.at[pl.ds(ebase, K)], didx_v)
        gather = pltpu.async_copy(h_hbm.at[sidx_v], rows_v, sem)
        for g in range(K // 16):
            si = sidx_v[pl.ds(g * 16, 16)]
            di = didx_v[pl.ds(g * 16, 16)]
            av = plsc.load_gather(tab_v, [si * 2])
            bv = plsc.load_gather(tab_v, [di * 2 + 1])
            e = av + bv
            e = jnp.maximum(e, 0.2 * e)
            w_v[pl.ds(g * 16, 16)] = jnp.exp(e)
        gather.wait()

        def _scale(j, _):
            wj = plsc.load_gather(w_v, [jnp.full((16,), j, jnp.int32)])
            for g in range(HID // 16):
                rows_v[j, pl.ds(g * 16, 16)] = rows_v[j, pl.ds(g * 16, 16)] * wj
            return 0
        lax.fori_loop(0, K, _scale, 0)

        pltpu.sync_copy(rows_v, out_sh.at[didx_v], add=True)
        pltpu.sync_copy(w_v, den_sh.at[didx_v], add=True)
        return 0

    lax.fori_loop(0, CHUNKS, _chunk, 0)
    plsc.subcore_barrier()

    for off in _stripe_offsets():
        sz = min(K, STRIPE - off)
        pltpu.sync_copy(out_sh.at[pl.ds(base + off, sz)],
                        outp_hbm.at[c, pl.ds(base + off, sz)])
    pltpu.sync_copy(den_sh.at[pl.ds(base, STRIPE)], denbuf_v)
    pltpu.sync_copy(denbuf_v, denp_hbm.at[pl.ds(c * NPAD + base, STRIPE)])


def _phase2(h, a_flat, srcp2d, dstp2d):
    mesh = plsc.VectorSubcoreMesh(core_axis_name="c", subcore_axis_name="s")
    fn = pl.kernel(
        _sc_body,
        out_type=[
            jax.ShapeDtypeStruct((NC, NPAD, HID), jnp.float32),
            jax.ShapeDtypeStruct((NC * NPAD,), jnp.float32),
        ],
        mesh=mesh,
        compiler_params=pltpu.CompilerParams(needs_layout_passes=False),
        scratch_types=[
            pltpu.VMEM((2 * N + 32,), jnp.float32),
            pltpu.VMEM((K,), jnp.int32),
            pltpu.VMEM((K,), jnp.int32),
            pltpu.VMEM((K,), jnp.float32),
            pltpu.VMEM((K, HID), jnp.float32),
            pltpu.VMEM((STRIPE,), jnp.float32),
            pltpu.VMEM_SHARED((NPAD, HID), jnp.float32),
            pltpu.VMEM_SHARED((NPAD,), jnp.float32),
            pltpu.SemaphoreType.DMA,
        ],
    )
    return fn(h, a_flat, srcp2d, dstp2d)


def _p3_body(op0_ref, op1_ref, dent_ref, a_ref, h_ref, b1_ref, w2_ref, b2_ref,
             o_ref):
    a_s = a_ref[:, 0]
    a_d = a_ref[:, 1]
    es = a_s + a_d
    es = jnp.maximum(es, 0.2 * es)
    w_self = jnp.exp(es)
    den = dent_ref[:, 0] + dent_ref[:, 1] + w_self + 1e-16
    out_un = op0_ref[...] + op1_ref[...] + w_self[:, None] * h_ref[...]
    h2 = out_un / den[:, None] + b1_ref[...]
    h2 = jnp.where(h2 > 0, h2, jnp.exp(h2) - 1.0)
    logits = jax.lax.dot_general(h2, w2_ref[...], (((1,), (0,)), ((), ())),
                                 preferred_element_type=jnp.float32)
    logits = logits + b2_ref[...]
    m = jnp.max(logits, axis=1, keepdims=True)
    z = logits - m
    o_ref[...] = z - jnp.log(jnp.sum(jnp.exp(z), axis=1, keepdims=True))


def _phase3(op0, op1, denT, a, h, b1, W2, b2):
    return pl.pallas_call(
        _p3_body,
        grid=(RB,),
        in_specs=[
            pl.BlockSpec((R, HID), lambda i: (i, 0)),
            pl.BlockSpec((R, HID), lambda i: (i, 0)),
            pl.BlockSpec((R, 2), lambda i: (i, 0)),
            pl.BlockSpec((R, 2), lambda i: (i, 0)),
            pl.BlockSpec((R, HID), lambda i: (i, 0)),
            pl.BlockSpec((1, HID), lambda i: (0, 0)),
            pl.BlockSpec((HID, OUT_C), lambda i: (0, 0)),
            pl.BlockSpec((1, OUT_C), lambda i: (0, 0)),
        ],
        out_specs=pl.BlockSpec((R, OUT_C), lambda i: (i, 0)),
        out_shape=jax.ShapeDtypeStruct((N, OUT_C), jnp.float32),
    )(op0, op1, denT, a, h, b1, W2, b2)


def kernel(x, edge_index, W1, att_src, att_dst, b1, W2, b2):
    att2 = jnp.stack([att_src, att_dst], axis=1)
    h, a = _phase1(x, W1, att2)

    pad = E_PAD - E
    srcp = jnp.concatenate([edge_index[0], jnp.zeros((pad,), jnp.int32)])
    dstp = jnp.concatenate([edge_index[1], jnp.full((pad,), N, jnp.int32)])

    outp, denp = _phase2(h, a.reshape(2 * N), srcp, dstp)

    denT = jnp.transpose(denp.reshape(NC, NPAD)[:, :N])
    return _phase3(outp[0, :N], outp[1, :N], denT, a, h,
                   b1.reshape(1, HID), W2, b2.reshape(1, OUT_C))


# speedup vs baseline: 17.9371x; 1.0763x over previous
import functools

import jax
import jax.numpy as jnp
from jax import lax
from jax.experimental import pallas as pl
from jax.experimental.pallas import tpu as pltpu
from jax.experimental.pallas import tpu_sc as plsc

N = 10000
E = 320000
IN_C = 128
HID = 128
OUT_C = 64

NC = 2
NS = 16
NW = NC * NS
K = 128
CHUNKS = 80
EPW = K * CHUNKS
E_PAD = EPW * NW
NPAD = 10112
STRIPE = NPAD // NS
RB = 25
R = N // RB


def _p1_body(x_ref, w1_ref, att2_ref, h_ref, a_ref):
    h = jax.lax.dot_general(x_ref[...], w1_ref[...], (((1,), (0,)), ((), ())),
                            preferred_element_type=jnp.float32)
    h_ref[...] = h
    a_ref[...] = jax.lax.dot_general(h, att2_ref[...], (((1,), (0,)), ((), ())),
                                     preferred_element_type=jnp.float32)


def _phase1(x, W1, att2):
    return pl.pallas_call(
        _p1_body,
        grid=(RB,),
        in_specs=[
            pl.BlockSpec((R, IN_C), lambda i: (i, 0)),
            pl.BlockSpec((IN_C, HID), lambda i: (0, 0)),
            pl.BlockSpec((HID, 2), lambda i: (0, 0)),
        ],
        out_specs=[
            pl.BlockSpec((R, HID), lambda i: (i, 0)),
            pl.BlockSpec((R, 2), lambda i: (i, 0)),
        ],
        out_shape=[
            jax.ShapeDtypeStruct((N, HID), jnp.float32),
            jax.ShapeDtypeStruct((N, 2), jnp.float32),
        ],
    )(x, W1, att2)


def _stripe_offsets():
    return list(range(0, STRIPE, K))


def _sc_body(h_hbm, a_hbm, src_hbm, dst_hbm,
             outp_hbm, denp_hbm,
             tab_v, sidx_v, didx_v, w_v, rows_v, denbuf_v,
             out_sh, den_sh,
             sem):
    c = lax.axis_index("c")
    s = lax.axis_index("s")
    wid = c * NS + s

    pltpu.sync_copy(a_hbm, tab_v.at[pl.ds(0, 2 * N)])
    z16 = jnp.zeros((16,), jnp.float32)
    tab_v[pl.ds(2 * N, 16)] = z16
    tab_v[pl.ds(2 * N + 16, 16)] = z16

    def _zrow(j, _):
        for g in range(HID // 16):
            rows_v[j, pl.ds(g * 16, 16)] = z16
        return 0
    lax.fori_loop(0, K, _zrow, 0)
    base = s * STRIPE
    for off in _stripe_offsets():
        sz = min(K, STRIPE - off)
        pltpu.sync_copy(rows_v.at[pl.ds(0, sz)], out_sh.at[pl.ds(base + off, sz)])
        pltpu.sync_copy(rows_v.at[0, pl.ds(0, sz)] if sz < K else rows_v.at[0],
                        den_sh.at[pl.ds(base + off, sz)])
    plsc.subcore_barrier()

    def _chunk(ci, _):
        ebase = wid * EPW + ci * K
        pltpu.sync_copy(src_hbm.at[pl.ds(ebase, K)], sidx_v)
        pltpu.sync_copy(dst_hbm.at[pl.ds(ebase, K)], didx_v)
        gather = pltpu.async_copy(h_hbm.at[sidx_v], rows_v, sem)
        for g in range(K // 16):
            si = sidx_v[pl.ds(g * 16, 16)]
            di = didx_v[pl.ds(g * 16, 16)]
            av = plsc.load_gather(tab_v, [si * 2])
            bv = plsc.load_gather(tab_v, [di * 2 + 1])
            e = av + bv
            e = jnp.maximum(e, 0.2 * e)
            w_v[pl.ds(g * 16, 16)] = jnp.exp(e)
        gather.wait()

        @plsc.parallel_loop(0, K, unroll=4)
        def _scale(j):
            wj = plsc.load_gather(w_v, [jnp.full((16,), j, jnp.int32)])
            for g in range(HID // 16):
                rows_v[j, pl.ds(g * 16, 16)] = rows_v[j, pl.ds(g * 16, 16)] * wj

        pltpu.sync_copy(rows_v, out_sh.at[didx_v], add=True)
        pltpu.sync_copy(w_v, den_sh.at[didx_v], add=True)
        return 0

    lax.fori_loop(0, CHUNKS, _chunk, 0)
    plsc.subcore_barrier()

    for off in _stripe_offsets():
        sz = min(K, STRIPE - off)
        pltpu.sync_copy(out_sh.at[pl.ds(base + off, sz)],
                        outp_hbm.at[c, pl.ds(base + off, sz)])
    pltpu.sync_copy(den_sh.at[pl.ds(base, STRIPE)], denbuf_v)
    pltpu.sync_copy(denbuf_v, denp_hbm.at[pl.ds(c * NPAD + base, STRIPE)])


def _phase2(h, a_flat, srcp2d, dstp2d):
    mesh = plsc.VectorSubcoreMesh(core_axis_name="c", subcore_axis_name="s")
    fn = pl.kernel(
        _sc_body,
        out_type=[
            jax.ShapeDtypeStruct((NC, NPAD, HID), jnp.float32),
            jax.ShapeDtypeStruct((NC * NPAD,), jnp.float32),
        ],
        mesh=mesh,
        compiler_params=pltpu.CompilerParams(needs_layout_passes=False),
        scratch_types=[
            pltpu.VMEM((2 * N + 32,), jnp.float32),
            pltpu.VMEM((K,), jnp.int32),
            pltpu.VMEM((K,), jnp.int32),
            pltpu.VMEM((K,), jnp.float32),
            pltpu.VMEM((K, HID), jnp.float32),
            pltpu.VMEM((STRIPE,), jnp.float32),
            pltpu.VMEM_SHARED((NPAD, HID), jnp.float32),
            pltpu.VMEM_SHARED((NPAD,), jnp.float32),
            pltpu.SemaphoreType.DMA,
        ],
    )
    return fn(h, a_flat, srcp2d, dstp2d)


def _p3_body(op0_ref, op1_ref, dent_ref, a_ref, h_ref, b1_ref, w2_ref, b2_ref,
             o_ref):
    a_s = a_ref[:, 0]
    a_d = a_ref[:, 1]
    es = a_s + a_d
    es = jnp.maximum(es, 0.2 * es)
    w_self = jnp.exp(es)
    den = dent_ref[:, 0] + dent_ref[:, 1] + w_self + 1e-16
    out_un = op0_ref[...] + op1_ref[...] + w_self[:, None] * h_ref[...]
    h2 = out_un / den[:, None] + b1_ref[...]
    h2 = jnp.where(h2 > 0, h2, jnp.exp(h2) - 1.0)
    logits = jax.lax.dot_general(h2, w2_ref[...], (((1,), (0,)), ((), ())),
                                 preferred_element_type=jnp.float32)
    logits = logits + b2_ref[...]
    m = jnp.max(logits, axis=1, keepdims=True)
    z = logits - m
    o_ref[...] = z - jnp.log(jnp.sum(jnp.exp(z), axis=1, keepdims=True))


def _phase3(op0, op1, denT, a, h, b1, W2, b2):
    return pl.pallas_call(
        _p3_body,
        grid=(RB,),
        in_specs=[
            pl.BlockSpec((R, HID), lambda i: (i, 0)),
            pl.BlockSpec((R, HID), lambda i: (i, 0)),
            pl.BlockSpec((R, 2), lambda i: (i, 0)),
            pl.BlockSpec((R, 2), lambda i: (i, 0)),
            pl.BlockSpec((R, HID), lambda i: (i, 0)),
            pl.BlockSpec((1, HID), lambda i: (0, 0)),
            pl.BlockSpec((HID, OUT_C), lambda i: (0, 0)),
            pl.BlockSpec((1, OUT_C), lambda i: (0, 0)),
        ],
        out_specs=pl.BlockSpec((R, OUT_C), lambda i: (i, 0)),
        out_shape=jax.ShapeDtypeStruct((N, OUT_C), jnp.float32),
    )(op0, op1, denT, a, h, b1, W2, b2)


def kernel(x, edge_index, W1, att_src, att_dst, b1, W2, b2):
    att2 = jnp.stack([att_src, att_dst], axis=1)
    h, a = _phase1(x, W1, att2)

    pad = E_PAD - E
    srcp = jnp.concatenate([edge_index[0], jnp.zeros((pad,), jnp.int32)])
    dstp = jnp.concatenate([edge_index[1], jnp.full((pad,), N, jnp.int32)])

    outp, denp = _phase2(h, a.reshape(2 * N), srcp, dstp)

    denT = jnp.transpose(denp.reshape(NC, NPAD)[:, :N])
    return _phase3(outp[0, :N], outp[1, :N], denT, a, h,
                   b1.reshape(1, HID), W2, b2.reshape(1, OUT_C))


# speedup vs baseline: 22.9752x; 1.2809x over previous
import functools

import jax
import jax.numpy as jnp
from jax import lax
from jax.experimental import pallas as pl
from jax.experimental.pallas import tpu as pltpu
from jax.experimental.pallas import tpu_sc as plsc

N = 10000
E = 320000
IN_C = 128
HID = 128
OUT_C = 64

NC = 2
NS = 16
NW = NC * NS
K = 64
CHUNKS = 160
EPW = K * CHUNKS
E_PAD = EPW * NW
NPAD = 10112
STRIPE = NPAD // NS
RB = 25
R = N // RB


def _p1_body(x_ref, w1_ref, att2_ref, h_ref, a_ref):
    h = jax.lax.dot_general(x_ref[...], w1_ref[...], (((1,), (0,)), ((), ())),
                            preferred_element_type=jnp.float32)
    h_ref[...] = h
    a_ref[...] = jax.lax.dot_general(h, att2_ref[...], (((1,), (0,)), ((), ())),
                                     preferred_element_type=jnp.float32)


def _phase1(x, W1, att2):
    return pl.pallas_call(
        _p1_body,
        grid=(RB,),
        in_specs=[
            pl.BlockSpec((R, IN_C), lambda i: (i, 0)),
            pl.BlockSpec((IN_C, HID), lambda i: (0, 0)),
            pl.BlockSpec((HID, 2), lambda i: (0, 0)),
        ],
        out_specs=[
            pl.BlockSpec((R, HID), lambda i: (i, 0)),
            pl.BlockSpec((R, 2), lambda i: (i, 0)),
        ],
        out_shape=[
            jax.ShapeDtypeStruct((N, HID), jnp.float32),
            jax.ShapeDtypeStruct((N, 2), jnp.float32),
        ],
    )(x, W1, att2)


def _sc_body(h_hbm, a_hbm, src_hbm, dst_hbm,
             outp_hbm, denp_hbm,
             tab_v, sidx_v, didx_v, w_v, rows_v, denbuf_v,
             out_sh, den_sh,
             sem0, sem1):
    c = lax.axis_index("c")
    s = lax.axis_index("s")
    wid = c * NS + s
    sems = (sem0, sem1)

    pltpu.sync_copy(a_hbm, tab_v.at[pl.ds(0, 2 * N)])
    z16 = jnp.zeros((16,), jnp.float32)
    tab_v[pl.ds(2 * N, 16)] = z16
    tab_v[pl.ds(2 * N + 16, 16)] = z16

    def _zrow(j, _):
        for b in range(2):
            for g in range(HID // 16):
                rows_v[b, j, pl.ds(g * 16, 16)] = z16
        return 0
    lax.fori_loop(0, K, _zrow, 0)
    base = s * STRIPE
    for off in range(0, STRIPE, K):
        sz = min(K, STRIPE - off)
        pltpu.sync_copy(rows_v.at[0, pl.ds(0, sz)],
                        out_sh.at[pl.ds(base + off, sz)])
        pltpu.sync_copy(rows_v.at[0, 0, pl.ds(0, sz)],
                        den_sh.at[pl.ds(base + off, sz)])
    plsc.subcore_barrier()

    def _load_idx(ci, b):
        ebase = wid * EPW + ci * K
        pltpu.sync_copy(src_hbm.at[pl.ds(ebase, K)], sidx_v.at[b])
        pltpu.sync_copy(dst_hbm.at[pl.ds(ebase, K)], didx_v.at[b])

    def _start_gather(b):
        return pltpu.async_copy(h_hbm.at[sidx_v.at[b]], rows_v.at[b], sems[b])

    _load_idx(0, 0)
    _start_gather(0)

    def _pair(t, _):
        for b in range(2):
            ci = 2 * t + b
            nb = 1 - b
            @pl.when(ci + 1 < CHUNKS)
            def _():
                _load_idx(ci + 1, nb)
                _start_gather(nb)
            for g in range(K // 16):
                si = sidx_v[b, pl.ds(g * 16, 16)]
                di = didx_v[b, pl.ds(g * 16, 16)]
                av = plsc.load_gather(tab_v, [si * 2])
                bv = plsc.load_gather(tab_v, [di * 2 + 1])
                e = av + bv
                e = jnp.maximum(e, 0.2 * e)
                w_v[b, pl.ds(g * 16, 16)] = jnp.exp(e)
            pltpu.make_async_copy(h_hbm.at[sidx_v.at[b]], rows_v.at[b],
                                  sems[b]).wait()

            @plsc.parallel_loop(0, K, unroll=4)
            def _scale(j):
                wj = plsc.load_gather(w_v.at[b], [jnp.full((16,), j, jnp.int32)])
                for g in range(HID // 16):
                    rows_v[b, j, pl.ds(g * 16, 16)] = (
                        rows_v[b, j, pl.ds(g * 16, 16)] * wj)

            pltpu.sync_copy(rows_v.at[b], out_sh.at[didx_v.at[b]], add=True)
            pltpu.sync_copy(w_v.at[b], den_sh.at[didx_v.at[b]], add=True)
        return 0

    lax.fori_loop(0, CHUNKS // 2, _pair, 0)
    plsc.subcore_barrier()

    for off in range(0, STRIPE, K):
        sz = min(K, STRIPE - off)
        pltpu.sync_copy(out_sh.at[pl.ds(base + off, sz)],
                        outp_hbm.at[c, pl.ds(base + off, sz)])
    pltpu.sync_copy(den_sh.at[pl.ds(base, STRIPE)], denbuf_v)
    pltpu.sync_copy(denbuf_v, denp_hbm.at[pl.ds(c * NPAD + base, STRIPE)])


def _phase2(h, a_flat, srcp2d, dstp2d):
    mesh = plsc.VectorSubcoreMesh(core_axis_name="c", subcore_axis_name="s")
    fn = pl.kernel(
        _sc_body,
        out_type=[
            jax.ShapeDtypeStruct((NC, NPAD, HID), jnp.float32),
            jax.ShapeDtypeStruct((NC * NPAD,), jnp.float32),
        ],
        mesh=mesh,
        compiler_params=pltpu.CompilerParams(needs_layout_passes=False),
        scratch_types=[
            pltpu.VMEM((2 * N + 32,), jnp.float32),
            pltpu.VMEM((2, K), jnp.int32),
            pltpu.VMEM((2, K), jnp.int32),
            pltpu.VMEM((2, K), jnp.float32),
            pltpu.VMEM((2, K, HID), jnp.float32),
            pltpu.VMEM((STRIPE,), jnp.float32),
            pltpu.VMEM_SHARED((NPAD, HID), jnp.float32),
            pltpu.VMEM_SHARED((NPAD,), jnp.float32),
            pltpu.SemaphoreType.DMA,
            pltpu.SemaphoreType.DMA,
        ],
    )
    return fn(h, a_flat, srcp2d, dstp2d)


def _p3_body(op0_ref, op1_ref, dent_ref, a_ref, h_ref, b1_ref, w2_ref, b2_ref,
             o_ref):
    a_s = a_ref[:, 0]
    a_d = a_ref[:, 1]
    es = a_s + a_d
    es = jnp.maximum(es, 0.2 * es)
    w_self = jnp.exp(es)
    den = dent_ref[:, 0] + dent_ref[:, 1] + w_self + 1e-16
    out_un = op0_ref[...] + op1_ref[...] + w_self[:, None] * h_ref[...]
    h2 = out_un / den[:, None] + b1_ref[...]
    h2 = jnp.where(h2 > 0, h2, jnp.exp(h2) - 1.0)
    logits = jax.lax.dot_general(h2, w2_ref[...], (((1,), (0,)), ((), ())),
                                 preferred_element_type=jnp.float32)
    logits = logits + b2_ref[...]
    m = jnp.max(logits, axis=1, keepdims=True)
    z = logits - m
    o_ref[...] = z - jnp.log(jnp.sum(jnp.exp(z), axis=1, keepdims=True))


def _phase3(op0, op1, denT, a, h, b1, W2, b2):
    return pl.pallas_call(
        _p3_body,
        grid=(RB,),
        in_specs=[
            pl.BlockSpec((R, HID), lambda i: (i, 0)),
            pl.BlockSpec((R, HID), lambda i: (i, 0)),
            pl.BlockSpec((R, 2), lambda i: (i, 0)),
            pl.BlockSpec((R, 2), lambda i: (i, 0)),
            pl.BlockSpec((R, HID), lambda i: (i, 0)),
            pl.BlockSpec((1, HID), lambda i: (0, 0)),
            pl.BlockSpec((HID, OUT_C), lambda i: (0, 0)),
            pl.BlockSpec((1, OUT_C), lambda i: (0, 0)),
        ],
        out_specs=pl.BlockSpec((R, OUT_C), lambda i: (i, 0)),
        out_shape=jax.ShapeDtypeStruct((N, OUT_C), jnp.float32),
    )(op0, op1, denT, a, h, b1, W2, b2)


def kernel(x, edge_index, W1, att_src, att_dst, b1, W2, b2):
    att2 = jnp.stack([att_src, att_dst], axis=1)
    h, a = _phase1(x, W1, att2)

    pad = E_PAD - E
    srcp = jnp.concatenate([edge_index[0], jnp.zeros((pad,), jnp.int32)])
    dstp = jnp.concatenate([edge_index[1], jnp.full((pad,), N, jnp.int32)])

    outp, denp = _phase2(h, a.reshape(2 * N), srcp, dstp)

    denT = jnp.transpose(denp.reshape(NC, NPAD)[:, :N])
    return _phase3(outp[0, :N], outp[1, :N], denT, a, h,
                   b1.reshape(1, HID), W2, b2.reshape(1, OUT_C))


# speedup vs baseline: 34.5748x; 1.5049x over previous
import functools

import jax
import jax.numpy as jnp
from jax import lax
from jax.experimental import pallas as pl
from jax.experimental.pallas import tpu as pltpu
from jax.experimental.pallas import tpu_sc as plsc

N = 10000
E = 320000
IN_C = 128
HID = 128
OUT_C = 64

NC = 2
NS = 16
NW = NC * NS
K = 80
CHUNKS = 125
EPW = K * CHUNKS
NPAD = 10112
STRIPE = NPAD // NS
RB = 25
R = N // RB


def _p1_body(x_ref, w1_ref, att2_ref, h_ref, a_ref):
    h = jax.lax.dot_general(x_ref[...], w1_ref[...], (((1,), (0,)), ((), ())),
                            preferred_element_type=jnp.float32)
    h_ref[...] = h
    a_ref[...] = jax.lax.dot_general(h, att2_ref[...], (((1,), (0,)), ((), ())),
                                     preferred_element_type=jnp.float32)


def _phase1(x, W1, att2):
    return pl.pallas_call(
        _p1_body,
        grid=(RB,),
        in_specs=[
            pl.BlockSpec((R, IN_C), lambda i: (i, 0)),
            pl.BlockSpec((IN_C, HID), lambda i: (0, 0)),
            pl.BlockSpec((HID, 2), lambda i: (0, 0)),
        ],
        out_specs=[
            pl.BlockSpec((R, HID), lambda i: (i, 0)),
            pl.BlockSpec((R, 2), lambda i: (i, 0)),
        ],
        out_shape=[
            jax.ShapeDtypeStruct((N, HID), jnp.float32),
            jax.ShapeDtypeStruct((N, 2), jnp.float32),
        ],
    )(x, W1, att2)


def _sc_body(h_hbm, a_hbm, src_hbm, dst_hbm,
             outp_hbm, denp_hbm,
             tab_v, sidx_v, didx_v, w_v, rows_v, denbuf_v,
             out_sh, den_sh,
             sem0, sem1):
    c = lax.axis_index("c")
    s = lax.axis_index("s")
    wid = c * NS + s
    sems = (sem0, sem1)

    pltpu.sync_copy(a_hbm, tab_v.at[pl.ds(0, 2 * N)])
    z16 = jnp.zeros((16,), jnp.float32)
    tab_v[pl.ds(2 * N, 16)] = z16
    tab_v[pl.ds(2 * N + 16, 16)] = z16

    def _zrow(j, _):
        for b in range(2):
            for g in range(HID // 16):
                rows_v[b, j, pl.ds(g * 16, 16)] = z16
        return 0
    lax.fori_loop(0, K, _zrow, 0)
    base = s * STRIPE
    for off in range(0, STRIPE, K):
        sz = min(K, STRIPE - off)
        pltpu.sync_copy(rows_v.at[0, pl.ds(0, sz)],
                        out_sh.at[pl.ds(base + off, sz)])
        pltpu.sync_copy(rows_v.at[0, 0, pl.ds(0, sz)],
                        den_sh.at[pl.ds(base + off, sz)])
    plsc.subcore_barrier()

    def _load_idx(ci, b):
        ebase = wid * EPW + ci * K
        pltpu.sync_copy(src_hbm.at[pl.ds(ebase, K)], sidx_v.at[b])
        pltpu.sync_copy(dst_hbm.at[pl.ds(ebase, K)], didx_v.at[b])

    def _start_gather(b):
        return pltpu.async_copy(h_hbm.at[sidx_v.at[b]], rows_v.at[b], sems[b])

    def _process(ci, b, stage_next):
        nb = 1 - b
        if stage_next:
            @pl.when(ci + 1 < CHUNKS)
            def _():
                _load_idx(ci + 1, nb)
                _start_gather(nb)
        for g in range(K // 16):
            si = sidx_v[b, pl.ds(g * 16, 16)]
            di = didx_v[b, pl.ds(g * 16, 16)]
            av = plsc.load_gather(tab_v, [si * 2])
            bv = plsc.load_gather(tab_v, [di * 2 + 1])
            e = av + bv
            e = jnp.maximum(e, 0.2 * e)
            w_v[b, pl.ds(g * 16, 16)] = jnp.exp(e)
        pltpu.make_async_copy(h_hbm.at[sidx_v.at[b]], rows_v.at[b],
                              sems[b]).wait()

        @plsc.parallel_loop(0, K, unroll=4)
        def _scale(j):
            wj = plsc.load_gather(w_v.at[b], [jnp.full((16,), j, jnp.int32)])
            for g in range(HID // 16):
                rows_v[b, j, pl.ds(g * 16, 16)] = (
                    rows_v[b, j, pl.ds(g * 16, 16)] * wj)

        pltpu.sync_copy(rows_v.at[b], out_sh.at[didx_v.at[b]], add=True)
        pltpu.sync_copy(w_v.at[b], den_sh.at[didx_v.at[b]], add=True)

    _load_idx(0, 0)
    _start_gather(0)

    def _pair(t, _):
        for b in range(2):
            _process(2 * t + b, b, True)
        return 0

    lax.fori_loop(0, CHUNKS // 2, _pair, 0)
    _process(CHUNKS - 1, (CHUNKS - 1) % 2, False)
    plsc.subcore_barrier()

    for off in range(0, STRIPE, K):
        sz = min(K, STRIPE - off)
        pltpu.sync_copy(out_sh.at[pl.ds(base + off, sz)],
                        outp_hbm.at[c, pl.ds(base + off, sz)])
    pltpu.sync_copy(den_sh.at[pl.ds(base, STRIPE)], denbuf_v)
    pltpu.sync_copy(denbuf_v, denp_hbm.at[pl.ds(c * NPAD + base, STRIPE)])


def _phase2(h, a_flat, srcp2d, dstp2d):
    mesh = plsc.VectorSubcoreMesh(core_axis_name="c", subcore_axis_name="s")
    fn = pl.kernel(
        _sc_body,
        out_type=[
            jax.ShapeDtypeStruct((NC, NPAD, HID), jnp.float32),
            jax.ShapeDtypeStruct((NC * NPAD,), jnp.float32),
        ],
        mesh=mesh,
        compiler_params=pltpu.CompilerParams(needs_layout_passes=False),
        scratch_types=[
            pltpu.VMEM((2 * N + 32,), jnp.float32),
            pltpu.VMEM((2, K), jnp.int32),
            pltpu.VMEM((2, K), jnp.int32),
            pltpu.VMEM((2, K), jnp.float32),
            pltpu.VMEM((2, K, HID), jnp.float32),
            pltpu.VMEM((STRIPE,), jnp.float32),
            pltpu.VMEM_SHARED((NPAD, HID), jnp.float32),
            pltpu.VMEM_SHARED((NPAD,), jnp.float32),
            pltpu.SemaphoreType.DMA,
            pltpu.SemaphoreType.DMA,
        ],
    )
    return fn(h, a_flat, srcp2d, dstp2d)


def _p3_body(op0_ref, op1_ref, dent_ref, a_ref, h_ref, b1_ref, w2_ref, b2_ref,
             o_ref):
    a_s = a_ref[:, 0]
    a_d = a_ref[:, 1]
    es = a_s + a_d
    es = jnp.maximum(es, 0.2 * es)
    w_self = jnp.exp(es)
    den = dent_ref[:, 0] + dent_ref[:, 1] + w_self + 1e-16
    out_un = op0_ref[...] + op1_ref[...] + w_self[:, None] * h_ref[...]
    h2 = out_un / den[:, None] + b1_ref[...]
    h2 = jnp.where(h2 > 0, h2, jnp.exp(h2) - 1.0)
    logits = jax.lax.dot_general(h2, w2_ref[...], (((1,), (0,)), ((), ())),
                                 preferred_element_type=jnp.float32)
    logits = logits + b2_ref[...]
    m = jnp.max(logits, axis=1, keepdims=True)
    z = logits - m
    o_ref[...] = z - jnp.log(jnp.sum(jnp.exp(z), axis=1, keepdims=True))


def _phase3(op0, op1, denT, a, h, b1, W2, b2):
    return pl.pallas_call(
        _p3_body,
        grid=(RB,),
        in_specs=[
            pl.BlockSpec((R, HID), lambda i: (i, 0)),
            pl.BlockSpec((R, HID), lambda i: (i, 0)),
            pl.BlockSpec((R, 2), lambda i: (i, 0)),
            pl.BlockSpec((R, 2), lambda i: (i, 0)),
            pl.BlockSpec((R, HID), lambda i: (i, 0)),
            pl.BlockSpec((1, HID), lambda i: (0, 0)),
            pl.BlockSpec((HID, OUT_C), lambda i: (0, 0)),
            pl.BlockSpec((1, OUT_C), lambda i: (0, 0)),
        ],
        out_specs=pl.BlockSpec((R, OUT_C), lambda i: (i, 0)),
        out_shape=jax.ShapeDtypeStruct((N, OUT_C), jnp.float32),
    )(op0, op1, denT, a, h, b1, W2, b2)


def kernel(x, edge_index, W1, att_src, att_dst, b1, W2, b2):
    att2 = jnp.stack([att_src, att_dst], axis=1)
    h, a = _phase1(x, W1, att2)

    outp, denp = _phase2(h, a.reshape(2 * N), edge_index[0], edge_index[1])

    denT = jnp.transpose(denp.reshape(NC, NPAD)[:, :N])
    return _phase3(outp[0, :N], outp[1, :N], denT, a, h,
                   b1.reshape(1, HID), W2, b2.reshape(1, OUT_C))


# speedup vs baseline: 35.7161x; 1.0330x over previous
import functools

import jax
import jax.numpy as jnp
from jax import lax
from jax.experimental import pallas as pl
from jax.experimental.pallas import tpu as pltpu
from jax.experimental.pallas import tpu_sc as plsc

N = 10000
E = 320000
IN_C = 128
HID = 128
OUT_C = 64

NC = 2
NS = 16
NW = NC * NS
K = 80
CHUNKS = 125
EPW = K * CHUNKS
NPAD = 10112
STRIPE = NPAD // NS
RB = 25
R = N // RB


def _p1_body(x_ref, w1_ref, att2_ref, h_ref, a_ref):
    h = jax.lax.dot_general(x_ref[...], w1_ref[...], (((1,), (0,)), ((), ())),
                            preferred_element_type=jnp.float32)
    h_ref[...] = h
    a_ref[...] = jax.lax.dot_general(h, att2_ref[...], (((1,), (0,)), ((), ())),
                                     preferred_element_type=jnp.float32)


def _phase1(x, W1, att2):
    return pl.pallas_call(
        _p1_body,
        grid=(RB,),
        in_specs=[
            pl.BlockSpec((R, IN_C), lambda i: (i, 0)),
            pl.BlockSpec((IN_C, HID), lambda i: (0, 0)),
            pl.BlockSpec((HID, 2), lambda i: (0, 0)),
        ],
        out_specs=[
            pl.BlockSpec((R, HID), lambda i: (i, 0)),
            pl.BlockSpec((R, 2), lambda i: (i, 0)),
        ],
        out_shape=[
            jax.ShapeDtypeStruct((N, HID), jnp.float32),
            jax.ShapeDtypeStruct((N, 2), jnp.float32),
        ],
    )(x, W1, att2)


def _sc_body(h_hbm, a_hbm, src_hbm, dst_hbm,
             outp_hbm, denp_hbm,
             tab_v, sidx_v, didx_v, w_v, rows_v, denbuf_v,
             out_sh, den_sh,
             sem0, sem1, rsem0, rsem1, wsem0, wsem1):
    c = lax.axis_index("c")
    s = lax.axis_index("s")
    wid = c * NS + s
    sems = (sem0, sem1)
    rsems = (rsem0, rsem1)
    wsems = (wsem0, wsem1)

    pltpu.sync_copy(a_hbm, tab_v.at[pl.ds(0, 2 * N)])
    z16 = jnp.zeros((16,), jnp.float32)
    tab_v[pl.ds(2 * N, 16)] = z16
    tab_v[pl.ds(2 * N + 16, 16)] = z16

    def _zrow(j, _):
        for b in range(2):
            for g in range(HID // 16):
                rows_v[b, j, pl.ds(g * 16, 16)] = z16
        return 0
    lax.fori_loop(0, K, _zrow, 0)
    base = s * STRIPE
    for off in range(0, STRIPE, K):
        sz = min(K, STRIPE - off)
        pltpu.sync_copy(rows_v.at[0, pl.ds(0, sz)],
                        out_sh.at[pl.ds(base + off, sz)])
        pltpu.sync_copy(rows_v.at[0, 0, pl.ds(0, sz)],
                        den_sh.at[pl.ds(base + off, sz)])
    plsc.subcore_barrier()

    def _load_idx(ci, b):
        ebase = wid * EPW + ci * K
        pltpu.sync_copy(src_hbm.at[pl.ds(ebase, K)], sidx_v.at[b])
        pltpu.sync_copy(dst_hbm.at[pl.ds(ebase, K)], didx_v.at[b])

    def _start_gather(b):
        return pltpu.async_copy(h_hbm.at[sidx_v.at[b]], rows_v.at[b], sems[b])

    def _process(ci, b, stage_next):
        nb = 1 - b
        if stage_next:
            @pl.when(ci + 1 < CHUNKS)
            def _():
                @pl.when(ci >= 1)
                def _():
                    pltpu.make_async_copy(
                        rows_v.at[nb], out_sh.at[didx_v.at[nb]],
                        rsems[nb]).wait()
                    pltpu.make_async_copy(
                        w_v.at[nb], den_sh.at[didx_v.at[nb]],
                        wsems[nb]).wait()
                _load_idx(ci + 1, nb)
                _start_gather(nb)
        for g in range(K // 16):
            si = sidx_v[b, pl.ds(g * 16, 16)]
            di = didx_v[b, pl.ds(g * 16, 16)]
            av = plsc.load_gather(tab_v, [si * 2])
            bv = plsc.load_gather(tab_v, [di * 2 + 1])
            e = av + bv
            e = jnp.maximum(e, 0.2 * e)
            w_v[b, pl.ds(g * 16, 16)] = jnp.exp(e)
        pltpu.make_async_copy(h_hbm.at[sidx_v.at[b]], rows_v.at[b],
                              sems[b]).wait()

        @plsc.parallel_loop(0, K, unroll=8)
        def _scale(j):
            wj = plsc.load_gather(w_v.at[b], [jnp.full((16,), j, jnp.int32)])
            for g in range(HID // 16):
                rows_v[b, j, pl.ds(g * 16, 16)] = (
                    rows_v[b, j, pl.ds(g * 16, 16)] * wj)

        pltpu.async_copy(rows_v.at[b], out_sh.at[didx_v.at[b]], rsems[b],
                         add=True)
        pltpu.async_copy(w_v.at[b], den_sh.at[didx_v.at[b]], wsems[b],
                         add=True)

    _load_idx(0, 0)
    _start_gather(0)

    def _pair(t, _):
        for b in range(2):
            _process(2 * t + b, b, True)
        return 0

    lax.fori_loop(0, CHUNKS // 2, _pair, 0)
    _process(CHUNKS - 1, (CHUNKS - 1) % 2, False)
    for b in range(2):
        pltpu.make_async_copy(rows_v.at[b], out_sh.at[didx_v.at[b]],
                              rsems[b]).wait()
        pltpu.make_async_copy(w_v.at[b], den_sh.at[didx_v.at[b]],
                              wsems[b]).wait()
    plsc.subcore_barrier()

    for off in range(0, STRIPE, K):
        sz = min(K, STRIPE - off)
        pltpu.sync_copy(out_sh.at[pl.ds(base + off, sz)],
                        outp_hbm.at[c, pl.ds(base + off, sz)])
    pltpu.sync_copy(den_sh.at[pl.ds(base, STRIPE)], denbuf_v)
    pltpu.sync_copy(denbuf_v, denp_hbm.at[pl.ds(c * NPAD + base, STRIPE)])


def _phase2(h, a_flat, srcp2d, dstp2d):
    mesh = plsc.VectorSubcoreMesh(core_axis_name="c", subcore_axis_name="s")
    fn = pl.kernel(
        _sc_body,
        out_type=[
            jax.ShapeDtypeStruct((NC, NPAD, HID), jnp.float32),
            jax.ShapeDtypeStruct((NC * NPAD,), jnp.float32),
        ],
        mesh=mesh,
        compiler_params=pltpu.CompilerParams(needs_layout_passes=False),
        scratch_types=[
            pltpu.VMEM((2 * N + 32,), jnp.float32),
            pltpu.VMEM((2, K), jnp.int32),
            pltpu.VMEM((2, K), jnp.int32),
            pltpu.VMEM((2, K), jnp.float32),
            pltpu.VMEM((2, K, HID), jnp.float32),
            pltpu.VMEM((STRIPE,), jnp.float32),
            pltpu.VMEM_SHARED((NPAD, HID), jnp.float32),
            pltpu.VMEM_SHARED((NPAD,), jnp.float32),
            pltpu.SemaphoreType.DMA,
            pltpu.SemaphoreType.DMA,
            pltpu.SemaphoreType.DMA,
            pltpu.SemaphoreType.DMA,
            pltpu.SemaphoreType.DMA,
            pltpu.SemaphoreType.DMA,
        ],
    )
    return fn(h, a_flat, srcp2d, dstp2d)


def _p3_body(op_ref, dent_ref, a_ref, h_ref, b1_ref, w2_ref, b2_ref,
             o_ref):
    a_s = a_ref[:, 0]
    a_d = a_ref[:, 1]
    es = a_s + a_d
    es = jnp.maximum(es, 0.2 * es)
    w_self = jnp.exp(es)
    den = dent_ref[:, 0] + dent_ref[:, 1] + w_self + 1e-16
    out_un = op_ref[0] + op_ref[1] + w_self[:, None] * h_ref[...]
    h2 = out_un / den[:, None] + b1_ref[...]
    h2 = jnp.where(h2 > 0, h2, jnp.exp(h2) - 1.0)
    logits = jax.lax.dot_general(h2, w2_ref[...], (((1,), (0,)), ((), ())),
                                 preferred_element_type=jnp.float32)
    logits = logits + b2_ref[...]
    m = jnp.max(logits, axis=1, keepdims=True)
    z = logits - m
    o_ref[...] = z - jnp.log(jnp.sum(jnp.exp(z), axis=1, keepdims=True))


def _phase3(op, denT, a, h, b1, W2, b2):
    return pl.pallas_call(
        _p3_body,
        grid=(RB,),
        in_specs=[
            pl.BlockSpec((NC, R, HID), lambda i: (0, i, 0)),
            pl.BlockSpec((R, 2), lambda i: (i, 0)),
            pl.BlockSpec((R, 2), lambda i: (i, 0)),
            pl.BlockSpec((R, HID), lambda i: (i, 0)),
            pl.BlockSpec((1, HID), lambda i: (0, 0)),
            pl.BlockSpec((HID, OUT_C), lambda i: (0, 0)),
            pl.BlockSpec((1, OUT_C), lambda i: (0, 0)),
        ],
        out_specs=pl.BlockSpec((R, OUT_C), lambda i: (i, 0)),
        out_shape=jax.ShapeDtypeStruct((N, OUT_C), jnp.float32),
    )(op, denT, a, h, b1, W2, b2)


def kernel(x, edge_index, W1, att_src, att_dst, b1, W2, b2):
    att2 = jnp.stack([att_src, att_dst], axis=1)
    h, a = _phase1(x, W1, att2)

    outp, denp = _phase2(h, a.reshape(2 * N), edge_index[0], edge_index[1])

    denT = jnp.transpose(denp.reshape(NC, NPAD)[:, :N])
    return _phase3(outp, denT, a, h,
                   b1.reshape(1, HID), W2, b2.reshape(1, OUT_C))
